# baseline (speedup 1.0000x reference)
import functools

import jax
import jax.numpy as jnp
from jax import lax
from jax.experimental import pallas as pl
from jax.experimental.pallas import tpu as pltpu
from jax.experimental.pallas import tpu_sc as plsc

_B, _S, _D = 4, 8192, 1024
_NC, _NS, _L = 2, 16, 16
_NW = _NC * _NS
_PW = _S // _NW
_R = 32
_NCHUNK = _PW // _R
_VPR = _D // _L


def _add_chunk(x_v, pos_v):
    def row_body(r, _):
        for u in range(_VPR):
            sl = pl.ds(u * _L, _L)
            plsc.addupdate(x_v.at[r, sl], pos_v[r, sl])
        return 0
    lax.fori_loop(0, _R, row_body, 0)


def _sc_body(x_hbm, pos_hbm, out_hbm, pos_v, xa_v, xb_v,
             sem_in_a, sem_in_b, sem_out_a, sem_out_b):
    wid = lax.axis_index("s") * _NC + lax.axis_index("c")
    p0 = wid * _PW
    bufs = (xa_v, xb_v)
    sems_in = (sem_in_a, sem_in_b)
    sems_out = (sem_out_a, sem_out_b)

    def chunk_body(c, _):
        row = p0 + c * _R
        h_in = {}
        h_out = {}
        h_in[0] = pltpu.async_copy(
            x_hbm.at[pl.ds(row, _R)], bufs[0], sems_in[0])
        pltpu.sync_copy(pos_hbm.at[pl.ds(row, _R)], pos_v)
        for b in range(_B):
            cur = b % 2
            if b > 0:
                h_out[b - 1].wait()
            if b + 1 < _B:
                nrow = (b + 1) * _S + row
                h_in[b + 1] = pltpu.async_copy(
                    x_hbm.at[pl.ds(nrow, _R)], bufs[1 - cur],
                    sems_in[1 - cur])
            h_in[b].wait()
            _add_chunk(bufs[cur], pos_v)
            h_out[b] = pltpu.async_copy(
                bufs[cur], out_hbm.at[pl.ds(b * _S + row, _R)],
                sems_out[cur])
        h_out[_B - 1].wait()
        return 0

    lax.fori_loop(0, _NCHUNK, chunk_body, 0)


def kernel(x, pos_table):
    B, S, D = x.shape
    x2 = x.reshape(B * S, D)
    mesh = plsc.VectorSubcoreMesh(core_axis_name="c", subcore_axis_name="s")
    k = functools.partial(
        pl.kernel,
        out_type=jax.ShapeDtypeStruct((B * S, D), jnp.float32),
        mesh=mesh,
        scratch_types=[
            pltpu.VMEM((_R, _D), jnp.float32),
            pltpu.VMEM((_R, _D), jnp.float32),
            pltpu.VMEM((_R, _D), jnp.float32),
            pltpu.SemaphoreType.DMA,
            pltpu.SemaphoreType.DMA,
            pltpu.SemaphoreType.DMA,
            pltpu.SemaphoreType.DMA,
        ],
    )(_sc_body)
    out = k(x2, pos_table)
    return out.reshape(B, S, D)


# speedup vs baseline: 3.4290x; 3.4290x over previous
import jax
import jax.numpy as jnp
from jax.experimental import pallas as pl


_TS = 2048


def _add_body(x_ref, pos_ref, out_ref):
    out_ref[...] = x_ref[...] + pos_ref[...][None, :, :]


def kernel(x, pos_table):
    B, S, D = x.shape
    n_s = S // _TS
    return pl.pallas_call(
        _add_body,
        grid=(n_s, B),
        in_specs=[
            pl.BlockSpec((1, _TS, D), lambda i, j: (j, i, 0)),
            pl.BlockSpec((_TS, D), lambda i, j: (i, 0)),
        ],
        out_specs=pl.BlockSpec((1, _TS, D), lambda i, j: (j, i, 0)),
        out_shape=jax.ShapeDtypeStruct((B, S, D), x.dtype),
    )(x, pos_table[:S])
